# Initial kernel scaffold; baseline (speedup 1.0000x reference)
import jax
import jax.numpy as jnp
from jax.experimental import pallas as pl


def kernel(tokens, emb_table, W1, b1, W2, b2):
    raise NotImplementedError("write your pallas kernel here")


# speedup vs baseline: 4.6552x; 4.6552x over previous
import functools

import jax
import jax.numpy as jnp
from jax import lax
from jax.experimental import pallas as pl
from jax.experimental.pallas import tpu as pltpu
from jax.experimental.pallas import tpu_sc as plsc

VOCAB_SIZE = 32000
EMB_DIM = 256
HID_DIM = 512
OUT_DIM = 384
BATCH = 4096
SEQ = 200

NUM_CORES = 2
NUM_SUBCORES = 16
NUM_WORKERS = NUM_CORES * NUM_SUBCORES
SAMPLES_PER_WORKER = BATCH // NUM_WORKERS
LANES = 16
NUM_VREGS = EMB_DIM // LANES
GATHER_A = 128
GATHER_B = SEQ - GATHER_A

_mesh = plsc.VectorSubcoreMesh(core_axis_name="c", subcore_axis_name="s")


@functools.partial(
    pl.kernel,
    out_type=jax.ShapeDtypeStruct((BATCH, EMB_DIM), jnp.float32),
    mesh=_mesh,
    scratch_types=[
        pltpu.VMEM((SAMPLES_PER_WORKER * SEQ,), jnp.int32),
        pltpu.VMEM((GATHER_A, EMB_DIM), jnp.float32),
        pltpu.VMEM((GATHER_B, EMB_DIM), jnp.float32),
        pltpu.VMEM((SAMPLES_PER_WORKER, EMB_DIM), jnp.float32),
    ],
)
def _pool(tokens_hbm, table_hbm, out_hbm, idx_v, buf_a, buf_b, acc_v):
    wid = lax.axis_index("s") * NUM_CORES + lax.axis_index("c")
    tok_base = wid * (SAMPLES_PER_WORKER * SEQ)
    pltpu.sync_copy(tokens_hbm.at[pl.ds(tok_base, SAMPLES_PER_WORKER * SEQ)],
                    idx_v)

    @pl.loop(0, SAMPLES_PER_WORKER)
    def _(s):
        off = pl.multiple_of(s * SEQ, 8)
        pltpu.sync_copy(table_hbm.at[idx_v.at[pl.ds(off, GATHER_A)]], buf_a)
        pltpu.sync_copy(table_hbm.at[idx_v.at[pl.ds(off + GATHER_A, GATHER_B)]],
                        buf_b)

        def add_rows(buf):
            def body(r, accs):
                return tuple(accs[c] + buf[r, pl.ds(c * LANES, LANES)]
                             for c in range(NUM_VREGS))
            return body

        zeros = tuple(jnp.zeros((LANES,), jnp.float32)
                      for _ in range(NUM_VREGS))
        accs = lax.fori_loop(0, GATHER_A, add_rows(buf_a), zeros)
        accs = lax.fori_loop(0, GATHER_B, add_rows(buf_b), accs)
        for c in range(NUM_VREGS):
            acc_v[s, pl.ds(c * LANES, LANES)] = accs[c]

    pltpu.sync_copy(acc_v, out_hbm.at[pl.ds(wid * SAMPLES_PER_WORKER,
                                            SAMPLES_PER_WORKER)])


_SQRT_HALF = 0.7071067811865476


def _mlp_kernel(x_ref, w1_ref, b1_ref, w2_ref, b2_ref, o_ref):
    x = x_ref[...]
    h = jnp.dot(x, w1_ref[...], preferred_element_type=jnp.float32)
    h = h * (1.0 / SEQ) + b1_ref[...]
    h = 0.5 * h * (1.0 + lax.erf(h * _SQRT_HALF))
    o_ref[...] = jnp.dot(h, w2_ref[...],
                         preferred_element_type=jnp.float32) + b2_ref[...]


def kernel(tokens, emb_table, W1, b1, W2, b2):
    pooled_sum = _pool(tokens.reshape(-1).astype(jnp.int32), emb_table)
    return pl.pallas_call(
        _mlp_kernel,
        out_shape=jax.ShapeDtypeStruct((BATCH, OUT_DIM), jnp.float32),
    )(pooled_sum, W1, b1.reshape(1, HID_DIM), W2, b2.reshape(1, OUT_DIM))


# speedup vs baseline: 9.6278x; 2.0682x over previous
import functools

import jax
import jax.numpy as jnp
from jax import lax
from jax.experimental import pallas as pl
from jax.experimental.pallas import tpu as pltpu
from jax.experimental.pallas import tpu_sc as plsc

VOCAB_SIZE = 32000
EMB_DIM = 256
HID_DIM = 512
OUT_DIM = 384
BATCH = 4096
SEQ = 200

NUM_CORES = 2
NUM_SUBCORES = 16
NUM_WORKERS = NUM_CORES * NUM_SUBCORES
SAMPLES_PER_WORKER = BATCH // NUM_WORKERS
LANES = 16
NUM_VREGS = EMB_DIM // LANES
GATHER_A = 128
GATHER_B = SEQ - GATHER_A

_mesh = plsc.VectorSubcoreMesh(core_axis_name="c", subcore_axis_name="s")


@functools.partial(
    pl.kernel,
    out_type=jax.ShapeDtypeStruct((BATCH, EMB_DIM), jnp.float32),
    mesh=_mesh,
    scratch_types=[
        pltpu.VMEM((SAMPLES_PER_WORKER * SEQ,), jnp.int32),
        pltpu.VMEM((SEQ, EMB_DIM), jnp.float32),
        pltpu.VMEM((SEQ, EMB_DIM), jnp.float32),
        pltpu.VMEM((EMB_DIM,), jnp.float32),
        pltpu.VMEM((EMB_DIM,), jnp.float32),
        pltpu.SemaphoreType.DMA,
        pltpu.SemaphoreType.DMA,
        pltpu.SemaphoreType.DMA,
        pltpu.SemaphoreType.DMA,
    ],
)
def _pool(tokens_hbm, table_hbm, out_hbm, idx_v, buf0, buf1, stage0, stage1,
          g0, g1, o0, o1):
    wid = lax.axis_index("s") * NUM_CORES + lax.axis_index("c")
    tok_base = wid * (SAMPLES_PER_WORKER * SEQ)
    row_base = wid * SAMPLES_PER_WORKER
    pltpu.sync_copy(tokens_hbm.at[pl.ds(tok_base, SAMPLES_PER_WORKER * SEQ)],
                    idx_v)

    def issue(s, buf, sem):
        off = pl.multiple_of(s * SEQ, 8)
        pltpu.async_copy(table_hbm.at[idx_v.at[pl.ds(off, GATHER_A)]],
                         buf.at[pl.ds(0, GATHER_A)], sem)
        pltpu.async_copy(
            table_hbm.at[idx_v.at[pl.ds(off + GATHER_A, GATHER_B)]],
            buf.at[pl.ds(GATHER_A, GATHER_B)], sem)

    def wait_gather(buf, sem):
        pltpu.make_async_copy(table_hbm.at[pl.ds(0, SEQ)], buf, sem).wait()

    def accumulate(buf):
        def body4(r, accs):
            for k in range(4):
                accs = tuple(accs[c] + buf[r * 4 + k, pl.ds(c * LANES, LANES)]
                             for c in range(NUM_VREGS))
            return accs
        zeros = tuple(jnp.zeros((LANES,), jnp.float32)
                      for _ in range(NUM_VREGS))
        return lax.fori_loop(0, SEQ // 4, body4, zeros)

    def emit(s, accs, stage, sem):
        @pl.when(s >= 2)
        def _():
            pltpu.make_async_copy(stage, out_hbm.at[row_base], sem).wait()
        for c in range(NUM_VREGS):
            stage[pl.ds(c * LANES, LANES)] = accs[c]
        pltpu.async_copy(stage, out_hbm.at[row_base + s], sem)

    issue(0, buf0, g0)
    issue(1, buf1, g1)

    @pl.loop(0, SAMPLES_PER_WORKER, step=2)
    def _(s):
        wait_gather(buf0, g0)
        accs = accumulate(buf0)

        @pl.when(s + 2 < SAMPLES_PER_WORKER)
        def _():
            issue(s + 2, buf0, g0)

        emit(s, accs, stage0, o0)

        wait_gather(buf1, g1)
        accs = accumulate(buf1)

        @pl.when(s + 3 < SAMPLES_PER_WORKER)
        def _():
            issue(s + 3, buf1, g1)

        emit(s + 1, accs, stage1, o1)

    pltpu.make_async_copy(stage0, out_hbm.at[row_base], o0).wait()
    pltpu.make_async_copy(stage1, out_hbm.at[row_base], o1).wait()


_SQRT_HALF = 0.7071067811865476


def _mlp_kernel(x_ref, w1_ref, b1_ref, w2_ref, b2_ref, o_ref):
    x = x_ref[...]
    h = jnp.dot(x, w1_ref[...], preferred_element_type=jnp.float32)
    h = h * (1.0 / SEQ) + b1_ref[...]
    h = 0.5 * h * (1.0 + lax.erf(h * _SQRT_HALF))
    o_ref[...] = jnp.dot(h, w2_ref[...],
                         preferred_element_type=jnp.float32) + b2_ref[...]


def kernel(tokens, emb_table, W1, b1, W2, b2):
    pooled_sum = _pool(tokens.reshape(-1).astype(jnp.int32), emb_table)
    return pl.pallas_call(
        _mlp_kernel,
        out_shape=jax.ShapeDtypeStruct((BATCH, OUT_DIM), jnp.float32),
    )(pooled_sum, W1, b1.reshape(1, HID_DIM), W2, b2.reshape(1, OUT_DIM))


# speedup vs baseline: 10.4301x; 1.0833x over previous
import functools

import jax
import jax.numpy as jnp
from jax import lax
from jax.experimental import pallas as pl
from jax.experimental.pallas import tpu as pltpu
from jax.experimental.pallas import tpu_sc as plsc

VOCAB_SIZE = 32000
EMB_DIM = 256
HID_DIM = 512
OUT_DIM = 384
BATCH = 4096
SEQ = 200

NUM_CORES = 2
NUM_SUBCORES = 16
NUM_WORKERS = NUM_CORES * NUM_SUBCORES
SAMPLES_PER_WORKER = BATCH // NUM_WORKERS
LANES = 16
NUM_VREGS = EMB_DIM // LANES
GATHER_A = 128
GATHER_B = SEQ - GATHER_A

_mesh = plsc.VectorSubcoreMesh(core_axis_name="c", subcore_axis_name="s")


@functools.partial(
    pl.kernel,
    out_type=jax.ShapeDtypeStruct((BATCH, EMB_DIM), jnp.float32),
    mesh=_mesh,
    scratch_types=[
        pltpu.VMEM((SAMPLES_PER_WORKER * SEQ,), jnp.int32),
        pltpu.VMEM((SEQ, EMB_DIM), jnp.float32),
        pltpu.VMEM((SEQ, EMB_DIM), jnp.float32),
        pltpu.VMEM((EMB_DIM,), jnp.float32),
        pltpu.VMEM((EMB_DIM,), jnp.float32),
        pltpu.SemaphoreType.DMA,
        pltpu.SemaphoreType.DMA,
        pltpu.SemaphoreType.DMA,
        pltpu.SemaphoreType.DMA,
    ],
)
def _pool(tokens_hbm, table_hbm, out_hbm, idx_v, buf0, buf1, stage0, stage1,
          g0, g1, o0, o1):
    wid = lax.axis_index("s") * NUM_CORES + lax.axis_index("c")
    tok_base = wid * (SAMPLES_PER_WORKER * SEQ)
    row_base = wid * SAMPLES_PER_WORKER
    pltpu.sync_copy(tokens_hbm.at[pl.ds(tok_base, SAMPLES_PER_WORKER * SEQ)],
                    idx_v)

    def issue(s, buf, sem):
        off = pl.multiple_of(s * SEQ, 8)
        pltpu.async_copy(table_hbm.at[idx_v.at[pl.ds(off, GATHER_A)]],
                         buf.at[pl.ds(0, GATHER_A)], sem)
        pltpu.async_copy(
            table_hbm.at[idx_v.at[pl.ds(off + GATHER_A, GATHER_B)]],
            buf.at[pl.ds(GATHER_A, GATHER_B)], sem)

    def accumulate(buf, sem):
        def body4(lo, hi, accs):
            def body(r, accs):
                for k in range(4):
                    accs = tuple(
                        accs[c] + buf[r * 4 + k, pl.ds(c * LANES, LANES)]
                        for c in range(NUM_VREGS))
                return accs
            return lax.fori_loop(lo // 4, hi // 4, body, accs)

        zeros = tuple(jnp.zeros((LANES,), jnp.float32)
                      for _ in range(NUM_VREGS))
        pltpu.make_async_copy(table_hbm.at[pl.ds(0, GATHER_A)],
                              buf.at[pl.ds(0, GATHER_A)], sem).wait()
        accs = body4(0, GATHER_A, zeros)
        pltpu.make_async_copy(table_hbm.at[pl.ds(0, GATHER_B)],
                              buf.at[pl.ds(GATHER_A, GATHER_B)], sem).wait()
        return body4(GATHER_A, SEQ, accs)

    def emit(s, accs, stage, sem):
        @pl.when(s >= 2)
        def _():
            pltpu.make_async_copy(stage, out_hbm.at[row_base], sem).wait()
        for c in range(NUM_VREGS):
            stage[pl.ds(c * LANES, LANES)] = accs[c]
        pltpu.async_copy(stage, out_hbm.at[row_base + s], sem)

    issue(0, buf0, g0)
    issue(1, buf1, g1)

    @pl.loop(0, SAMPLES_PER_WORKER, step=2)
    def _(s):
        accs = accumulate(buf0, g0)

        @pl.when(s + 2 < SAMPLES_PER_WORKER)
        def _():
            issue(s + 2, buf0, g0)

        emit(s, accs, stage0, o0)

        accs = accumulate(buf1, g1)

        @pl.when(s + 3 < SAMPLES_PER_WORKER)
        def _():
            issue(s + 3, buf1, g1)

        emit(s + 1, accs, stage1, o1)

    pltpu.make_async_copy(stage0, out_hbm.at[row_base], o0).wait()
    pltpu.make_async_copy(stage1, out_hbm.at[row_base], o1).wait()


_SQRT_HALF = 0.7071067811865476


def _mlp_kernel(x_ref, w1_ref, b1_ref, w2_ref, b2_ref, o_ref):
    x = x_ref[...]
    h = jnp.dot(x, w1_ref[...], preferred_element_type=jnp.float32)
    h = h * (1.0 / SEQ) + b1_ref[...]
    h = 0.5 * h * (1.0 + lax.erf(h * _SQRT_HALF))
    o_ref[...] = jnp.dot(h, w2_ref[...],
                         preferred_element_type=jnp.float32) + b2_ref[...]


def kernel(tokens, emb_table, W1, b1, W2, b2):
    pooled_sum = _pool(tokens.reshape(-1).astype(jnp.int32), emb_table)
    return pl.pallas_call(
        _mlp_kernel,
        out_shape=jax.ShapeDtypeStruct((BATCH, OUT_DIM), jnp.float32),
    )(pooled_sum, W1, b1.reshape(1, HID_DIM), W2, b2.reshape(1, OUT_DIM))
